# baseline (speedup 1.0000x reference)
import functools

import jax
import jax.numpy as jnp
from jax import lax
from jax.experimental import pallas as pl
from jax.experimental.pallas import tpu as pltpu
from jax.experimental.pallas import tpu_sc as plsc

N = 10000
E = 320000
D = 128
H = 64

NC = 2
NS = 16
NT = NC * NS
HALF_N = 5120
TRASH = HALF_N - N // 2
CHUNK = 512
STEPS = 40
E_PAD = NS * STEPS * CHUNK


REGION = E_PAD // NS
RROWS = REGION // CHUNK


def _sc_part_body(src_hbm, dst_hbm, tsrc_hbm, tdst_hbm, zd_hbm, ones_hbm,
                  psrc_hbm, pdst_hbm, pcnt_hbm, dout_hbm,
                  sv, dv, osrc, odst, pcv, onesv, dacc, dsem0, dsem1):
  cid = lax.axis_index("c")
  sid = lax.axis_index("s")
  lo = cid * (N // 2)
  rz = sid * (HALF_N // NS)
  pltpu.sync_copy(zd_hbm.at[pl.ds(rz, HALF_N // NS)],
                  dacc.at[pl.ds(rz, HALF_N // NS)])
  pltpu.sync_copy(ones_hbm, onesv)
  pltpu.sync_copy(src_hbm.at[pl.ds(sid * REGION, REGION)], sv)
  pltpu.sync_copy(dst_hbm.at[pl.ds(sid * REGION, REGION)], dv)
  pltpu.sync_copy(tsrc_hbm, osrc.at[pl.ds(0, REGION)])
  pltpu.sync_copy(tdst_hbm, odst.at[pl.ds(0, REGION)])

  iota16 = lax.iota(jnp.int32, 16)

  def grp(k, off):
    s16 = sv[pl.ds(k * 16, 16)]
    d16 = dv[pl.ds(k * 16, 16)]
    m = (d16 >= lo) & (d16 < lo + N // 2)
    key = 1 - m.astype(jnp.int32)
    packed = s16 * 8192 + (d16 - lo)
    _, vs = plsc.sort_key_val(key, packed)
    pos = off + iota16
    plsc.store_scatter(osrc, [pos], lax.shift_right_arithmetic(vs, 13))
    plsc.store_scatter(odst, [pos], vs & 8191)
    return off + plsc.all_reduce_population_count(m)

  cnt = lax.fori_loop(0, REGION // 16, grp, jnp.zeros((16,), jnp.int32))
  plsc.store_scatter(osrc, [cnt + iota16], jnp.zeros((16,), jnp.int32))
  plsc.store_scatter(odst, [cnt + iota16], N // 2 + iota16)
  pcv[...] = jnp.maximum((cnt + 2 * CHUNK - 1) // (2 * CHUNK), 1)
  pltpu.sync_copy(osrc.at[pl.ds(0, REGION)], psrc_hbm.at[cid, sid])
  pltpu.sync_copy(odst.at[pl.ds(0, REGION)], pdst_hbm.at[cid, sid])
  pltpu.sync_copy(pcv, pcnt_hbm.at[cid, sid])

  plsc.subcore_barrier()
  dsems = (dsem0, dsem1)

  def ddrain(b):
    pltpu.make_async_copy(
        onesv, dacc.at[odst.at[pl.ds(b * CHUNK, CHUNK)]], dsems[b]).wait()

  def dstep(g, carry):
    for b in (0, 1):
      s = g * 2 + b
      @pl.when(g >= 1)
      def _():
        ddrain(b)
      pltpu.async_copy(onesv, dacc.at[odst.at[pl.ds(s * CHUNK, CHUNK)]],
                       dsems[b], add=True)
    return carry

  lax.fori_loop(0, RROWS // 2, dstep, 0)
  ddrain(0)
  ddrain(1)
  plsc.subcore_barrier()
  pltpu.sync_copy(dacc.at[pl.ds(rz, HALF_N // NS)],
                  dout_hbm.at[cid, pl.ds(rz, HALF_N // NS)])


def _sc_agg_body(y_hbm, psrc_hbm, pdst_hbm, pcnt_hbm, zh_hbm, out_hbm,
                 srcv, dstv, rows, cntv, ysp, acc, gsem, ssem0, ssem1):
  ssems = (ssem0, ssem1)
  cid = lax.axis_index("c")
  sid = lax.axis_index("s")
  rz = sid * (HALF_N // NS)
  pltpu.sync_copy(zh_hbm.at[pl.ds(rz, HALF_N // NS)],
                  acc.at[pl.ds(rz, HALF_N // NS)])
  pltpu.sync_copy(y_hbm.at[pl.ds(sid * (N // NS), N // NS)],
                  ysp.at[pl.ds(sid * (N // NS), N // NS)])
  pltpu.sync_copy(pcnt_hbm.at[cid, sid], cntv)
  plsc.subcore_barrier()

  def drain(b):
    pltpu.make_async_copy(rows.at[b], acc.at[dstv.at[b]], ssems[b]).wait()

  def half_step(g, b):
    s = g * 2 + b
    @pl.when(g >= 1)
    def _():
      drain(b)
    pltpu.sync_copy(psrc_hbm.at[cid, sid, pl.ds(s * CHUNK, CHUNK)],
                    srcv.at[b])
    pltpu.sync_copy(pdst_hbm.at[cid, sid, pl.ds(s * CHUNK, CHUNK)],
                    dstv.at[b])
    pltpu.async_copy(ysp.at[srcv.at[b]], rows.at[b], gsem).wait()
    pltpu.async_copy(rows.at[b], acc.at[dstv.at[b]], ssems[b], add=True)

  nchp = cntv[...][0]

  def step(g, carry):
    @pl.when(g < nchp)
    def _():
      half_step(g, 0)
      half_step(g, 1)
    return carry

  lax.fori_loop(0, STEPS // 2, step, 0)
  drain(0)
  drain(1)
  plsc.subcore_barrier()
  pltpu.sync_copy(acc.at[pl.ds(rz, HALF_N // NS)],
                  out_hbm.at[cid, pl.ds(rz, HALF_N // NS)])


@functools.lru_cache(maxsize=None)
def _get_sc_part():
  mesh = plsc.VectorSubcoreMesh(core_axis_name="c", subcore_axis_name="s")
  return pl.kernel(
      _sc_part_body,
      out_type=(jax.ShapeDtypeStruct((NC, NS, REGION), jnp.int32),
                jax.ShapeDtypeStruct((NC, NS, REGION), jnp.int32),
                jax.ShapeDtypeStruct((NC, NS, 16), jnp.int32),
                jax.ShapeDtypeStruct((NC, HALF_N, 16), jnp.float32)),
      mesh=mesh,
      scratch_types=[
          pltpu.VMEM((REGION,), jnp.int32),
          pltpu.VMEM((REGION,), jnp.int32),
          pltpu.VMEM((REGION + 16,), jnp.int32),
          pltpu.VMEM((REGION + 16,), jnp.int32),
          pltpu.VMEM((16,), jnp.int32),
          pltpu.VMEM((CHUNK, 16), jnp.float32),
          pltpu.VMEM_SHARED((HALF_N, 16), jnp.float32),
          pltpu.SemaphoreType.DMA,
          pltpu.SemaphoreType.DMA,
      ],
      compiler_params=pltpu.CompilerParams(use_tc_tiling_on_sc=False,
                                           needs_layout_passes=False))


@functools.lru_cache(maxsize=None)
def _get_sc_agg():
  mesh = plsc.VectorSubcoreMesh(core_axis_name="c", subcore_axis_name="s")
  return pl.kernel(
      _sc_agg_body,
      out_type=(jax.ShapeDtypeStruct((NC, HALF_N, H), jnp.float32),),
      mesh=mesh,
      scratch_types=[
          pltpu.VMEM((2, CHUNK), jnp.int32),
          pltpu.VMEM((2, CHUNK), jnp.int32),
          pltpu.VMEM((2, CHUNK, H), jnp.float32),
          pltpu.VMEM((16,), jnp.int32),
          pltpu.VMEM_SHARED((N, H), jnp.float32),
          pltpu.VMEM_SHARED((HALF_N, H), jnp.float32),
          pltpu.SemaphoreType.DMA,
          pltpu.SemaphoreType.DMA,
          pltpu.SemaphoreType.DMA,
      ],
      compiler_params=pltpu.CompilerParams(use_tc_tiling_on_sc=False))


def _sc_agg(*args):
  return _get_sc_agg()(*args)


def _sc_part(*args):
  return _get_sc_part()(*args)


_BR = 2000


def _dot_t(a, w):
  return lax.dot_general(a, w, (((1,), (1,)), ((), ())),
                         preferred_element_type=jnp.float32)


def _pre_kernel(x_ref, wl_ref, wr_ref, y_ref, r_ref):
  xb = x_ref[...]
  y_ref[...] = _dot_t(xb, wl_ref[...])
  r_ref[...] = _dot_t(xb, wr_ref[...])


def _tc_pre(x, wl, wr):
  n, d = x.shape
  h = wl.shape[0]
  return pl.pallas_call(
      _pre_kernel,
      grid=(n // _BR,),
      in_specs=[pl.BlockSpec((_BR, d), lambda i: (i, 0)),
                pl.BlockSpec((h, d), lambda i: (0, 0)),
                pl.BlockSpec((h, d), lambda i: (0, 0))],
      out_specs=[pl.BlockSpec((_BR, h), lambda i: (i, 0)),
                 pl.BlockSpec((_BR, h), lambda i: (i, 0))],
      out_shape=[jax.ShapeDtypeStruct((n, h), jnp.float32),
                 jax.ShapeDtypeStruct((n, h), jnp.float32)],
  )(x, wl, wr)


_BR2 = 1000
_PB2 = (N // 2) // _BR2


def _p_map(i):
  return (i // _PB2, i % _PB2, 0)


def _mean_term(p_ref, d_ref):
  deg = d_ref[0][:, :1]
  return p_ref[0] / jnp.maximum(deg, 1.0)


def _make_mid_kernel(with_res):
  def kern(p, dp, b, rc, *rest):
    if with_res:
      res, wl, wr, ho, yo, ro = rest
    else:
      wl, wr, ho, yo, ro = rest
    m = _mean_term(p, dp) + b[...] + rc[...]
    hh = jnp.maximum(m, 0.0)
    if with_res:
      hh = hh + res[...]
    ho[...] = hh
    yo[...] = _dot_t(hh, wl[...])
    ro[...] = _dot_t(hh, wr[...])
  return kern


def _tc_mid(p, dp, b, rc, res, wl, wr):
  with_res = res is not None
  in_specs = [
      pl.BlockSpec((1, _BR2, H), _p_map),
      pl.BlockSpec((1, _BR2, 16), _p_map),
      pl.BlockSpec((1, H), lambda i: (0, 0)),
      pl.BlockSpec((_BR2, H), lambda i: (i, 0)),
  ]
  args = [p, dp, b, rc]
  if with_res:
    in_specs.append(pl.BlockSpec((_BR2, H), lambda i: (i, 0)))
    args.append(res)
  in_specs += [pl.BlockSpec((H, H), lambda i: (0, 0)),
               pl.BlockSpec((H, H), lambda i: (0, 0))]
  args += [wl, wr]
  return pl.pallas_call(
      _make_mid_kernel(with_res),
      grid=(N // _BR2,),
      in_specs=in_specs,
      out_specs=[pl.BlockSpec((_BR2, H), lambda i: (i, 0))] * 3,
      out_shape=[jax.ShapeDtypeStruct((N, H), jnp.float32)] * 3,
  )(*args)


def _final_kernel(p, dp, b, rc, out):
  o = _mean_term(p, dp) + b[...] + rc[...]
  nrm = jnp.sqrt(jnp.sum(o * o, axis=1, keepdims=True))
  out[...] = o / jnp.maximum(nrm, 1e-12)


def _tc_final(p, dp, b, rc):
  return pl.pallas_call(
      _final_kernel,
      grid=(N // _BR2,),
      in_specs=[
          pl.BlockSpec((1, _BR2, H), _p_map),
          pl.BlockSpec((1, _BR2, 16), _p_map),
          pl.BlockSpec((1, H), lambda i: (0, 0)),
          pl.BlockSpec((_BR2, H), lambda i: (i, 0)),
      ],
      out_specs=pl.BlockSpec((_BR2, H), lambda i: (i, 0)),
      out_shape=jax.ShapeDtypeStruct((N, H), jnp.float32),
  )(p, dp, b, rc)


def kernel(x, edge_index, W1l, b1l, W1r, W2l, b2l, W2r,
           W3l, b3l, W3r, W4l, b4l, W4r):
  src = edge_index[0]
  dst = edge_index[1]
  pad = E_PAD - E
  src2 = jnp.concatenate(
      [src, jnp.zeros((pad,), jnp.int32)]).reshape(E_PAD // CHUNK, CHUNK)
  dstraw = jnp.concatenate(
      [dst, jnp.full((pad,), -1, jnp.int32)]).reshape(E_PAD // CHUNK, CHUNK)
  half = N // 2
  tsrc = jnp.zeros((REGION,), jnp.int32)
  tdst = half + (jnp.arange(REGION, dtype=jnp.int32) % TRASH)
  zh = jnp.zeros((HALF_N, H), jnp.float32)
  zd = jnp.zeros((HALF_N, 16), jnp.float32)
  ones16 = jnp.ones((CHUNK, 16), jnp.float32)
  b1 = b1l.reshape(1, H)
  b2 = b2l.reshape(1, H)
  b3 = b3l.reshape(1, H)
  b4 = b4l.reshape(1, H)

  psrc, pdst, pcnt, dp = _sc_part(src2.reshape(-1), dstraw.reshape(-1),
                                  tsrc, tdst, zd, ones16)
  y1, r1 = _tc_pre(x, W1l, W1r)
  (p1,) = _sc_agg(y1, psrc, pdst, pcnt, zh)
  h1, y2, r2 = _tc_mid(p1, dp, b1, r1, None, W2l, W2r)
  (p2,) = _sc_agg(y2, psrc, pdst, pcnt, zh)
  h2, y3, r3 = _tc_mid(p2, dp, b2, r2, h1, W3l, W3r)
  (p3,) = _sc_agg(y3, psrc, pdst, pcnt, zh)
  h3, y4, r4 = _tc_mid(p3, dp, b3, r3, h2, W4l, W4r)
  (p4,) = _sc_agg(y4, psrc, pdst, pcnt, zh)
  return _tc_final(p4, dp, b4, r4)
